# baseline (speedup 1.0000x reference)
import jax
import jax.numpy as jnp
from jax.experimental import pallas as pl
from jax.experimental.pallas import tpu as pltpu

_VMEM_LIMIT = 100 * 1024 * 1024
_ENC_TILE = 256
_DEC_BANDS = 4


def _gae_kernel(x_ref, adj_ref, w0_ref, w1_ref, out_ref,
                t_ref, adjb_ref, u_ref, z_ref):
    g = pl.program_id(0)
    n = adjb_ref.shape[0]
    tm = adj_ref.shape[0]
    nenc = n // tm

    @pl.when(g == 0)
    def _():
        x = x_ref[...].astype(jnp.bfloat16)
        w0 = w0_ref[...].astype(jnp.bfloat16)
        t_ref[...] = jnp.dot(
            x, w0, preferred_element_type=jnp.float32
        ).astype(jnp.bfloat16)

    @pl.when(g < nenc)
    def _():
        adj_b = adj_ref[...].astype(jnp.bfloat16)
        adjb_ref[pl.ds(g * tm, tm), :] = adj_b
        h = jnp.dot(adj_b, t_ref[...], preferred_element_type=jnp.float32)
        h = jnp.maximum(h, 0.0).astype(jnp.bfloat16)
        w1 = w1_ref[...].astype(jnp.bfloat16)
        u_ref[pl.ds(g * tm, tm), :] = jnp.dot(
            h, w1, preferred_element_type=jnp.float32
        ).astype(jnp.bfloat16)

    @pl.when(g == nenc - 1)
    def _():
        z_ref[...] = jnp.dot(
            adjb_ref[...], u_ref[...], preferred_element_type=jnp.float32
        ).astype(jnp.bfloat16)

    @pl.when(g >= nenc)
    def _():
        band = out_ref.shape[0]
        row = (g - nenc) * band
        zr = z_ref[pl.ds(row, band), :]
        logits = jax.lax.dot_general(
            zr, z_ref[...],
            dimension_numbers=(((1,), (1,)), ((), ())),
            preferred_element_type=jnp.float32,
        )
        out_ref[...] = jax.nn.sigmoid(logits)


@jax.jit
def kernel(x, adj, w0, w1):
    n, in_dim = x.shape
    h1 = w0.shape[1]
    h2 = w1.shape[1]

    tm = _ENC_TILE if n % _ENC_TILE == 0 else n
    nenc = n // tm
    band = n // _DEC_BANDS if n % _DEC_BANDS == 0 else n
    ndec = n // band
    nsteps = nenc + ndec

    a_pred = pl.pallas_call(
        _gae_kernel,
        out_shape=jax.ShapeDtypeStruct((n, n), jnp.float32),
        grid=(nsteps,),
        in_specs=[
            pl.BlockSpec((n, in_dim), lambda g: (0, 0)),
            pl.BlockSpec((tm, n), lambda g: (jnp.minimum(g, nenc - 1), 0)),
            pl.BlockSpec((in_dim, h1), lambda g: (0, 0)),
            pl.BlockSpec((h1, h2), lambda g: (0, 0)),
        ],
        out_specs=pl.BlockSpec(
            (band, n), lambda g: (jnp.maximum(g - nenc, 0), 0)
        ),
        scratch_shapes=[
            pltpu.VMEM((n, h1), jnp.bfloat16),
            pltpu.VMEM((n, n), jnp.bfloat16),
            pltpu.VMEM((n, h2), jnp.bfloat16),
            pltpu.VMEM((n, h2), jnp.bfloat16),
        ],
        compiler_params=pltpu.CompilerParams(
            dimension_semantics=("arbitrary",),
            vmem_limit_bytes=_VMEM_LIMIT,
        ),
    )(x, adj, w0, w1)

    return a_pred
